# baseline (speedup 1.0000x reference)
import functools

import jax
import jax.numpy as jnp
from jax import lax
from jax.experimental import pallas as pl
from jax.experimental.pallas import tpu as pltpu
from jax.experimental.pallas import tpu_sc as plsc

N = 10000
E = 320000
NC, NS = 2, 16
CH = 80
NPAD = NC * NS * 640
BM = 400
NBLK = N // BM
F32 = jnp.float32

_MESH = plsc.VectorSubcoreMesh(
    core_axis_name="c", subcore_axis_name="s", num_cores=NC, num_subcores=NS
)


def _deg_body(dst_hbm, out_hbm, idx_v, ones_v, zbuf_v, acc_sh):
    c = lax.axis_index("c")
    s = lax.axis_index("s")
    z16 = jnp.zeros((16,), F32)
    for i in range(640 // 16):
        zbuf_v[pl.ds(i * 16, 16)] = z16
    o16 = jnp.full((16,), 1.0, F32)
    for i in range(CH // 16):
        ones_v[pl.ds(i * 16, 16)] = o16
    pltpu.sync_copy(zbuf_v, acc_sh.at[pl.ds(s * 640, 640)])
    plsc.subcore_barrier()
    per = E // (NC * NS)
    base = (c * NS + s) * per

    def body(j, carry):
        pltpu.sync_copy(dst_hbm.at[pl.ds(base + j * CH, CH)], idx_v)
        pltpu.sync_copy(ones_v, acc_sh.at[idx_v], add=True)
        return carry

    lax.fori_loop(0, per // CH, body, 0)
    plsc.subcore_barrier()
    pltpu.sync_copy(acc_sh.at[pl.ds(s * 640, 640)], out_hbm.at[c, pl.ds(s * 640, 640)])


_deg_call = pl.kernel(
    _deg_body,
    out_type=jax.ShapeDtypeStruct((NC, NPAD), F32),
    mesh=_MESH,
    scratch_types=[
        pltpu.VMEM((CH,), jnp.int32),
        pltpu.VMEM((CH,), F32),
        pltpu.VMEM((640,), F32),
        pltpu.VMEM_SHARED((NPAD,), F32),
    ],
)


PCH = 32
NBUF = 8
ED_GROUPS = 40
CH_GROUPS = 80
NACC = N + 16


def _make_prop(split_channels):
    ngroups = CH_GROUPS if split_channels else ED_GROUPS

    def body(u_hbm, src_hbm, dst_hbm, out_hbm, sidx_v, didx_v, rows_v, zbuf_v,
             acc_sh, isems, gsems, ssems):
        c = lax.axis_index("c")
        s = lax.axis_index("s")

        def sidx_src(t):
            return src_hbm.at[c, s, t] if split_channels else src_hbm.at[c * NS + s, t]

        def didx_src(t):
            return dst_hbm.at[s, t] if split_channels else dst_hbm.at[c * NS + s, t]

        def idx_load(t, q):
            pltpu.make_async_copy(sidx_src(t), sidx_v.at[q], isems.at[q]).start()
            pltpu.make_async_copy(didx_src(t), didx_v.at[q], isems.at[q]).start()

        def idx_wait(t, q):
            pltpu.make_async_copy(sidx_src(t), sidx_v.at[q], isems.at[q]).wait()
            pltpu.make_async_copy(didx_src(t), didx_v.at[q], isems.at[q]).wait()

        def gather(q, p):
            pltpu.make_async_copy(
                u_hbm.at[sidx_v.at[q, p]], rows_v.at[p], gsems.at[p]).start()

        def gather_wait(q, p):
            pltpu.make_async_copy(
                u_hbm.at[sidx_v.at[q, p]], rows_v.at[p], gsems.at[p]).wait()

        def scatter(q, p):
            pltpu.async_copy(
                rows_v.at[p], acc_sh.at[didx_v.at[q, p]], ssems.at[p], add=True)

        def scatter_wait(q, p):
            pltpu.make_async_copy(
                rows_v.at[p], acc_sh.at[didx_v.at[q, p]], ssems.at[p]).wait()

        idx_load(0, 0)

        z16 = jnp.zeros((16,), F32)
        for r in range(16):
            for i in range(8):
                zbuf_v[r, pl.ds(i * 16, 16)] = z16

        def zb(k, carry):
            pltpu.sync_copy(zbuf_v, acc_sh.at[pl.ds(s * 624 + k * 16, 16)])
            return carry

        lax.fori_loop(0, 39, zb, 0)

        @pl.when(s == NS - 1)
        def _():
            pltpu.sync_copy(zbuf_v, acc_sh.at[pl.ds(N - 16, 16)])
            pltpu.sync_copy(zbuf_v, acc_sh.at[pl.ds(N, 16)])

        idx_wait(0, 0)
        idx_load(1, 1)
        for p in range(NBUF):
            gather(0, p)
        plsc.subcore_barrier()

        def group(t, q, q1):
            for p in range(NBUF):
                gather_wait(q, p)
                scatter(q, p)

            @pl.when(t + 1 < ngroups)
            def _():
                idx_wait(t + 1, q1)

            for p in range(NBUF):
                scatter_wait(q, p)

                @pl.when(t + 1 < ngroups)
                def _():
                    gather(q1, p)

            @pl.when(t + 2 < ngroups)
            def _():
                idx_load(t + 2, q)

        def grp2(m, carry):
            group(2 * m, 0, 1)
            group(2 * m + 1, 1, 0)
            return carry

        lax.fori_loop(0, ngroups // 2, grp2, 0)
        plsc.subcore_barrier()
        pltpu.sync_copy(acc_sh.at[pl.ds(s * 624, 624)], out_hbm.at[c, pl.ds(s * 624, 624)])

        @pl.when(s == NS - 1)
        def _():
            pltpu.sync_copy(acc_sh.at[pl.ds(N - 16, 16)], out_hbm.at[c, pl.ds(N - 16, 16)])

    return pl.kernel(
        body,
        out_type=jax.ShapeDtypeStruct((NC, N, 128), F32),
        mesh=_MESH,
        scratch_types=[
            pltpu.VMEM((2, NBUF, PCH), jnp.int32),
            pltpu.VMEM((2, NBUF, PCH), jnp.int32),
            pltpu.VMEM((NBUF, PCH, 128), F32),
            pltpu.VMEM((16, 128), F32),
            pltpu.VMEM_SHARED((NACC, 128), F32),
            pltpu.SemaphoreType.DMA((2,)),
            pltpu.SemaphoreType.DMA((NBUF,)),
            pltpu.SemaphoreType.DMA((NBUF,)),
        ],
        name="prop_ch" if split_channels else "prop_ed",
    )


_prop_ch = _make_prop(True)
_prop_ed = _make_prop(False)


def _prep_body(x_ref, wg_ref, degp_ref, ux_ref, dinv_ref, gates_ref, loss_ref,
               acc_imp, acc_load):
    i = pl.program_id(0)
    deg = degp_ref[:, 0] + degp_ref[:, 1] + 1.0
    dinv = lax.rsqrt(deg)[:, None]
    dinv_ref[...] = dinv
    ux_ref[...] = x_ref[...] * dinv
    logits = jnp.dot(x_ref[...], wg_ref[...], preferred_element_type=F32)
    cols = lax.broadcasted_iota(jnp.int32, (BM, 8), 1)
    remaining = jnp.ones((BM, 8), jnp.bool_)
    chosen = jnp.zeros((BM, 8), jnp.bool_)
    for _ in range(4):
        cur = jnp.where(remaining, logits, -jnp.inf)
        m = jnp.max(cur, axis=1, keepdims=True)
        first = jnp.min(jnp.where(cur == m, cols, 8), axis=1, keepdims=True)
        pick = cols == first
        chosen = jnp.logical_or(chosen, pick)
        remaining = jnp.logical_and(remaining, jnp.logical_not(pick))
    m1 = jnp.max(logits, axis=1, keepdims=True)
    ex = jnp.where(chosen, jnp.exp(logits - m1), 0.0)
    gates = ex / jnp.sum(ex, axis=1, keepdims=True)
    gates_ref[...] = gates

    @pl.when(i == 0)
    def _():
        acc_imp[...] = jnp.zeros_like(acc_imp)
        acc_load[...] = jnp.zeros_like(acc_load)

    acc_imp[...] += jnp.sum(gates, axis=0, keepdims=True)
    acc_load[...] += jnp.sum((gates > 0).astype(F32), axis=0, keepdims=True)

    @pl.when(i == NBLK - 1)
    def _():
        def cv2(v):
            mean = jnp.sum(v) / 8.0
            var = jnp.sum((v - mean) ** 2) / 7.0
            return var / (mean * mean + 1e-10)

        val = (cv2(acc_imp[...]) + cv2(acc_load[...])) * 0.001
        loss_ref[...] = jnp.broadcast_to(val, (1, 1))


def _prep(x, w_gate, degp):
    return pl.pallas_call(
        _prep_body,
        grid=(NBLK,),
        in_specs=[
            pl.BlockSpec((BM, 128), lambda i: (i, 0)),
            pl.BlockSpec((128, 8), lambda i: (0, 0)),
            pl.BlockSpec((BM, 2), lambda i: (i, 0)),
        ],
        out_specs=[
            pl.BlockSpec((BM, 128), lambda i: (i, 0)),
            pl.BlockSpec((BM, 1), lambda i: (i, 0)),
            pl.BlockSpec((BM, 8), lambda i: (i, 0)),
            pl.BlockSpec((1, 1), lambda i: (0, 0)),
        ],
        out_shape=[
            jax.ShapeDtypeStruct((N, 128), F32),
            jax.ShapeDtypeStruct((N, 1), F32),
            jax.ShapeDtypeStruct((N, 8), F32),
            jax.ShapeDtypeStruct((1, 1), F32),
        ],
        scratch_shapes=[pltpu.VMEM((1, 8), F32), pltpu.VMEM((1, 8), F32)],
    )(x, w_gate, degp)


def _fl_rest_body(sq_ref, ux_ref, dinv_ref, w0_ref, w1_ref, out_ref):
    dinv = dinv_ref[...]
    q = dinv * (sq_ref[0] + sq_ref[1] + ux_ref[...])
    h1 = jax.nn.relu(jnp.dot(q, w0_ref[0], preferred_element_type=F32))
    u = dinv * jnp.dot(h1, w1_ref[0], preferred_element_type=F32)
    out_ref[0, 0] = u[:, :128]
    out_ref[0, 1] = u[:, 128:]


def _fl_rest(sq, ux, dinv, w0s, w1s):
    return pl.pallas_call(
        _fl_rest_body,
        grid=(7, NBLK),
        in_specs=[
            pl.BlockSpec((2, BM, 128), lambda e, i: (0, i, 0)),
            pl.BlockSpec((BM, 128), lambda e, i: (i, 0)),
            pl.BlockSpec((BM, 1), lambda e, i: (i, 0)),
            pl.BlockSpec((1, 128, 256), lambda e, i: (e, 0, 0)),
            pl.BlockSpec((1, 256, 256), lambda e, i: (e, 0, 0)),
        ],
        out_specs=pl.BlockSpec((1, 2, BM, 128), lambda e, i: (e, 0, i, 0)),
        out_shape=jax.ShapeDtypeStruct((7, 2, N, 128), F32),
    )(sq, ux, dinv, w0s, w1s)


def _fl_e0_body(sq_ref, ux_ref, dinv_ref, w0_ref, w1_ref, out_ref):
    dinv = dinv_ref[...]
    q = dinv * (sq_ref[0] + sq_ref[1] + ux_ref[...])
    h1 = jax.nn.relu(jnp.dot(q, w0_ref[...], preferred_element_type=F32))
    out_ref[...] = dinv * jnp.dot(h1, w1_ref[...], preferred_element_type=F32)


def _fl_e0(sq, ux, dinv, w0, w1):
    return pl.pallas_call(
        _fl_e0_body,
        grid=(NBLK,),
        in_specs=[
            pl.BlockSpec((2, BM, 128), lambda i: (0, i, 0)),
            pl.BlockSpec((BM, 128), lambda i: (i, 0)),
            pl.BlockSpec((BM, 1), lambda i: (i, 0)),
            pl.BlockSpec((128, 256), lambda i: (0, 0)),
            pl.BlockSpec((256, 128), lambda i: (0, 0)),
        ],
        out_specs=pl.BlockSpec((BM, 128), lambda i: (i, 0)),
        out_shape=jax.ShapeDtypeStruct((N, 128), F32),
    )(sq, ux, dinv, w0, w1)


def _mid_body(s_ref, u_ref, dinv_ref, w_ref, out_ref):
    dinv = dinv_ref[...]
    t0 = jax.nn.relu(dinv * (s_ref[0] + u_ref[0]))
    t1 = jax.nn.relu(dinv * (s_ref[1] + u_ref[1]))
    w = w_ref[...]
    z = jnp.dot(t0, w[:128, :], preferred_element_type=F32)
    z = z + jnp.dot(t1, w[128:, :], preferred_element_type=F32)
    u = dinv * z
    out_ref[0] = u[:, :128]
    out_ref[1] = u[:, 128:]


def _mid(s, u, dinv, w):
    return pl.pallas_call(
        _mid_body,
        grid=(NBLK,),
        in_specs=[
            pl.BlockSpec((2, BM, 128), lambda i: (0, i, 0)),
            pl.BlockSpec((2, BM, 128), lambda i: (0, i, 0)),
            pl.BlockSpec((BM, 1), lambda i: (i, 0)),
            pl.BlockSpec((256, 256), lambda i: (0, 0)),
        ],
        out_specs=pl.BlockSpec((2, BM, 128), lambda i: (0, i, 0)),
        out_shape=jax.ShapeDtypeStruct((2, N, 128), F32),
    )(s, u, dinv, w)


def _last_body(s_ref, u_ref, dinv_ref, w_ref, out_ref):
    dinv = dinv_ref[...]
    t0 = jax.nn.relu(dinv * (s_ref[0] + u_ref[0]))
    t1 = jax.nn.relu(dinv * (s_ref[1] + u_ref[1]))
    w = w_ref[...]
    z = jnp.dot(t0, w[:128, :], preferred_element_type=F32)
    z = z + jnp.dot(t1, w[128:, :], preferred_element_type=F32)
    out_ref[...] = dinv * z


def _last(s, u, dinv, w):
    return pl.pallas_call(
        _last_body,
        grid=(NBLK,),
        in_specs=[
            pl.BlockSpec((2, BM, 128), lambda i: (0, i, 0)),
            pl.BlockSpec((2, BM, 128), lambda i: (0, i, 0)),
            pl.BlockSpec((BM, 1), lambda i: (i, 0)),
            pl.BlockSpec((256, 128), lambda i: (0, 0)),
        ],
        out_specs=pl.BlockSpec((BM, 128), lambda i: (i, 0)),
        out_shape=jax.ShapeDtypeStruct((N, 128), F32),
    )(s, u, dinv, w)


def _combine_body(gates_ref, dinv_ref, *refs):
    s_refs = refs[:8]
    u_refs = refs[8:16]
    y_ref = refs[16]
    dinv = dinv_ref[...]
    gates = gates_ref[...]
    y = jnp.zeros((BM, 128), F32)
    for e in range(8):
        out_e = dinv * (s_refs[e][0] + s_refs[e][1] + u_refs[e][...])
        y = y + gates[:, e:e + 1] * out_e
    y_ref[...] = y


def _combine(gates, dinv, s_list, u_list):
    return pl.pallas_call(
        _combine_body,
        grid=(NBLK,),
        in_specs=(
            [pl.BlockSpec((BM, 8), lambda i: (i, 0)),
             pl.BlockSpec((BM, 1), lambda i: (i, 0))]
            + [pl.BlockSpec((2, BM, 128), lambda i: (0, i, 0)) for _ in range(8)]
            + [pl.BlockSpec((BM, 128), lambda i: (i, 0)) for _ in range(8)]
        ),
        out_specs=pl.BlockSpec((BM, 128), lambda i: (i, 0)),
        out_shape=jax.ShapeDtypeStruct((N, 128), F32),
    )(gates, dinv, *s_list, *u_list)


def kernel(x, adj_t, w_gate, expert_weights):
    src = adj_t[0]
    dst = adj_t[1]
    pe = ED_GROUPS * NBUF * PCH - E // (NC * NS)
    pc = CH_GROUPS * NBUF * PCH - E // NS
    ps = (jnp.arange(pc, dtype=jnp.int32) * 131) % N
    pd = N + (jnp.arange(pc, dtype=jnp.int32) % 16)
    src_ed = jnp.concatenate(
        [src.reshape(NC * NS, -1), jnp.broadcast_to(ps[:pe], (NC * NS, pe))],
        axis=1).reshape(NC * NS, ED_GROUPS, NBUF, PCH)
    dst_ed = jnp.concatenate(
        [dst.reshape(NC * NS, -1), jnp.broadcast_to(pd[:pe], (NC * NS, pe))],
        axis=1).reshape(NC * NS, ED_GROUPS, NBUF, PCH)
    base_s = jnp.concatenate(
        [src.reshape(NS, -1), jnp.broadcast_to(ps, (NS, pc))], axis=1)
    src_ch = jnp.stack([base_s, base_s + N]).reshape(2, NS, CH_GROUPS, NBUF, PCH)
    dst_ch = jnp.concatenate(
        [dst.reshape(NS, -1), jnp.broadcast_to(pd, (NS, pc))], axis=1
    ).reshape(NS, CH_GROUPS, NBUF, PCH)

    degp = _deg_call(dst)[:, :N].T
    ux, dinv, gates, loss11 = _prep(x, w_gate, degp)

    sq = _prop_ed(ux, src_ed, dst_ed)

    w0s = jnp.stack([expert_weights[e][0] for e in range(1, 8)])
    w1s = jnp.stack([expert_weights[e][1] for e in range(1, 8)])
    u1_rest = _fl_rest(sq, ux, dinv, w0s, w1s)
    u0 = _fl_e0(sq, ux, dinv, expert_weights[0][0], expert_weights[0][1])

    s_list = [None] * 8
    u_list = [None] * 8
    s_list[0] = _prop_ed(u0, src_ed, dst_ed)
    u_list[0] = u0
    for e in range(1, 8):
        L = 2 + e
        u = u1_rest[e - 1]
        ul = None
        for l in range(1, L - 1):
            sprop = _prop_ch(u.reshape(NC * N, 128), src_ch, dst_ch)
            w_next = expert_weights[e][l + 1]
            if l + 1 <= L - 2:
                u = _mid(sprop, u, dinv, w_next)
            else:
                ul = _last(sprop, u, dinv, w_next)
        s_list[e] = _prop_ed(ul, src_ed, dst_ed)
        u_list[e] = ul

    y = _combine(gates, dinv, s_list, u_list)
    return (y, loss11.reshape(()))


# speedup vs baseline: 1.0519x; 1.0519x over previous
import functools

import jax
import jax.numpy as jnp
from jax import lax
from jax.experimental import pallas as pl
from jax.experimental.pallas import tpu as pltpu
from jax.experimental.pallas import tpu_sc as plsc

N = 10000
E = 320000
NC, NS = 2, 16
CH = 80
NPAD = NC * NS * 640
BM = 400
NBLK = N // BM
F32 = jnp.float32

_MESH = plsc.VectorSubcoreMesh(
    core_axis_name="c", subcore_axis_name="s", num_cores=NC, num_subcores=NS
)


PCH = 64
NBUF = 4
ED_GROUPS = 40
CH_GROUPS = 80
NACC = N + 16


def _make_prop(split_channels):
    ngroups = CH_GROUPS if split_channels else ED_GROUPS

    def body(u_hbm, src_hbm, dst_hbm, out_hbm, sidx_v, didx_v, rows_v, zbuf_v,
             acc_sh, isems, gsems, ssems):
        c = lax.axis_index("c")
        s = lax.axis_index("s")

        def sidx_src(t):
            return src_hbm.at[c, s, t] if split_channels else src_hbm.at[c * NS + s, t]

        def didx_src(t):
            return dst_hbm.at[s, t] if split_channels else dst_hbm.at[c * NS + s, t]

        def idx_load(t, q):
            pltpu.make_async_copy(sidx_src(t), sidx_v.at[q], isems.at[q]).start()
            pltpu.make_async_copy(didx_src(t), didx_v.at[q], isems.at[q]).start()

        def idx_wait(t, q):
            pltpu.make_async_copy(sidx_src(t), sidx_v.at[q], isems.at[q]).wait()
            pltpu.make_async_copy(didx_src(t), didx_v.at[q], isems.at[q]).wait()

        def gather(q, p):
            pltpu.make_async_copy(
                u_hbm.at[sidx_v.at[q, p]], rows_v.at[p], gsems.at[p]).start()

        def gather_wait(q, p):
            pltpu.make_async_copy(
                u_hbm.at[sidx_v.at[q, p]], rows_v.at[p], gsems.at[p]).wait()

        def scatter(q, p):
            pltpu.async_copy(
                rows_v.at[p], acc_sh.at[didx_v.at[q, p]], ssems.at[p], add=True)

        def scatter_wait(q, p):
            pltpu.make_async_copy(
                rows_v.at[p], acc_sh.at[didx_v.at[q, p]], ssems.at[p]).wait()

        idx_load(0, 0)

        z16 = jnp.zeros((16,), F32)
        for r in range(16):
            for i in range(8):
                zbuf_v[r, pl.ds(i * 16, 16)] = z16

        def zb(k, carry):
            pltpu.sync_copy(zbuf_v, acc_sh.at[pl.ds(s * 624 + k * 16, 16)])
            return carry

        lax.fori_loop(0, 39, zb, 0)

        @pl.when(s == NS - 1)
        def _():
            pltpu.sync_copy(zbuf_v, acc_sh.at[pl.ds(N - 16, 16)])
            pltpu.sync_copy(zbuf_v, acc_sh.at[pl.ds(N, 16)])

        idx_wait(0, 0)
        idx_load(1, 1)
        for p in range(NBUF):
            gather(0, p)
        plsc.subcore_barrier()

        def group(t, q, q1):
            for p in range(NBUF):
                gather_wait(q, p)
                scatter(q, p)

            @pl.when(t + 1 < ngroups)
            def _():
                idx_wait(t + 1, q1)

            for p in range(NBUF):
                scatter_wait(q, p)

                @pl.when(t + 1 < ngroups)
                def _():
                    gather(q1, p)

            @pl.when(t + 2 < ngroups)
            def _():
                idx_load(t + 2, q)

        def grp2(m, carry):
            group(2 * m, 0, 1)
            group(2 * m + 1, 1, 0)
            return carry

        lax.fori_loop(0, ngroups // 2, grp2, 0)
        plsc.subcore_barrier()
        pltpu.sync_copy(acc_sh.at[pl.ds(s * 624, 624)], out_hbm.at[c, pl.ds(s * 624, 624)])

        @pl.when(s == NS - 1)
        def _():
            pltpu.sync_copy(acc_sh.at[pl.ds(N - 16, 16)], out_hbm.at[c, pl.ds(N - 16, 16)])

    return pl.kernel(
        body,
        out_type=jax.ShapeDtypeStruct((NC, N, 128), F32),
        mesh=_MESH,
        scratch_types=[
            pltpu.VMEM((2, NBUF, PCH), jnp.int32),
            pltpu.VMEM((2, NBUF, PCH), jnp.int32),
            pltpu.VMEM((NBUF, PCH, 128), F32),
            pltpu.VMEM((16, 128), F32),
            pltpu.VMEM_SHARED((NACC, 128), F32),
            pltpu.SemaphoreType.DMA((2,)),
            pltpu.SemaphoreType.DMA((NBUF,)),
            pltpu.SemaphoreType.DMA((NBUF,)),
        ],
        name="prop_ch" if split_channels else "prop_ed",
    )


_prop_ch = _make_prop(True)
_prop_ed = _make_prop(False)


def _deg_body(dst_hbm, out_hbm, didx_v, ones_v, zbuf_v, acc_sh, isems, ssems):
    c = lax.axis_index("c")
    s = lax.axis_index("s")
    w = c * NS + s

    def idx_load(t, q):
        pltpu.make_async_copy(dst_hbm.at[w, t], didx_v.at[q], isems.at[q]).start()

    def idx_wait(t, q):
        pltpu.make_async_copy(dst_hbm.at[w, t], didx_v.at[q], isems.at[q]).wait()

    def scatter(q, p):
        pltpu.async_copy(ones_v, acc_sh.at[didx_v.at[q, p]], ssems.at[p], add=True)

    def scatter_wait(q, p):
        pltpu.make_async_copy(ones_v, acc_sh.at[didx_v.at[q, p]], ssems.at[p]).wait()

    idx_load(0, 0)
    z16 = jnp.zeros((16,), F32)
    for i in range(640 // 16):
        zbuf_v[pl.ds(i * 16, 16)] = z16
    o16 = jnp.full((16,), 1.0, F32)
    for i in range(PCH // 16):
        ones_v[pl.ds(i * 16, 16)] = o16
    pltpu.sync_copy(zbuf_v, acc_sh.at[pl.ds(s * 640, 640)])
    idx_wait(0, 0)
    idx_load(1, 1)
    plsc.subcore_barrier()

    def group(t, q, q1):
        for p in range(NBUF):
            scatter(q, p)

        @pl.when(t + 1 < ED_GROUPS)
        def _():
            idx_wait(t + 1, q1)

        for p in range(NBUF):
            scatter_wait(q, p)

        @pl.when(t + 2 < ED_GROUPS)
        def _():
            idx_load(t + 2, q)

    def grp2(m, carry):
        group(2 * m, 0, 1)
        group(2 * m + 1, 1, 0)
        return carry

    lax.fori_loop(0, ED_GROUPS // 2, grp2, 0)
    plsc.subcore_barrier()
    pltpu.sync_copy(acc_sh.at[pl.ds(s * 640, 640)], out_hbm.at[c, pl.ds(s * 640, 640)])


_deg_call = pl.kernel(
    _deg_body,
    out_type=jax.ShapeDtypeStruct((NC, NPAD), F32),
    mesh=_MESH,
    scratch_types=[
        pltpu.VMEM((2, NBUF, PCH), jnp.int32),
        pltpu.VMEM((PCH,), F32),
        pltpu.VMEM((640,), F32),
        pltpu.VMEM_SHARED((NPAD,), F32),
        pltpu.SemaphoreType.DMA((2,)),
        pltpu.SemaphoreType.DMA((NBUF,)),
    ],
    name="deg",
)


def _prep_body(x_ref, wg_ref, degp_ref, ux_ref, dinv_ref, gates_ref, loss_ref,
               acc_imp, acc_load):
    i = pl.program_id(0)
    deg = degp_ref[:, 0] + degp_ref[:, 1] + 1.0
    dinv = lax.rsqrt(deg)[:, None]
    dinv_ref[...] = dinv
    ux_ref[...] = x_ref[...] * dinv
    logits = jnp.dot(x_ref[...], wg_ref[...], preferred_element_type=F32)
    cols = lax.broadcasted_iota(jnp.int32, (BM, 8), 1)
    remaining = jnp.ones((BM, 8), jnp.bool_)
    chosen = jnp.zeros((BM, 8), jnp.bool_)
    for _ in range(4):
        cur = jnp.where(remaining, logits, -jnp.inf)
        m = jnp.max(cur, axis=1, keepdims=True)
        first = jnp.min(jnp.where(cur == m, cols, 8), axis=1, keepdims=True)
        pick = cols == first
        chosen = jnp.logical_or(chosen, pick)
        remaining = jnp.logical_and(remaining, jnp.logical_not(pick))
    m1 = jnp.max(logits, axis=1, keepdims=True)
    ex = jnp.where(chosen, jnp.exp(logits - m1), 0.0)
    gates = ex / jnp.sum(ex, axis=1, keepdims=True)
    gates_ref[...] = gates

    @pl.when(i == 0)
    def _():
        acc_imp[...] = jnp.zeros_like(acc_imp)
        acc_load[...] = jnp.zeros_like(acc_load)

    acc_imp[...] += jnp.sum(gates, axis=0, keepdims=True)
    acc_load[...] += jnp.sum((gates > 0).astype(F32), axis=0, keepdims=True)

    @pl.when(i == NBLK - 1)
    def _():
        def cv2(v):
            mean = jnp.sum(v) / 8.0
            var = jnp.sum((v - mean) ** 2) / 7.0
            return var / (mean * mean + 1e-10)

        val = (cv2(acc_imp[...]) + cv2(acc_load[...])) * 0.001
        loss_ref[...] = jnp.broadcast_to(val, (1, 1))


def _prep(x, w_gate, degp):
    return pl.pallas_call(
        _prep_body,
        grid=(NBLK,),
        in_specs=[
            pl.BlockSpec((BM, 128), lambda i: (i, 0)),
            pl.BlockSpec((128, 8), lambda i: (0, 0)),
            pl.BlockSpec((BM, 2), lambda i: (i, 0)),
        ],
        out_specs=[
            pl.BlockSpec((BM, 128), lambda i: (i, 0)),
            pl.BlockSpec((BM, 1), lambda i: (i, 0)),
            pl.BlockSpec((BM, 8), lambda i: (i, 0)),
            pl.BlockSpec((1, 1), lambda i: (0, 0)),
        ],
        out_shape=[
            jax.ShapeDtypeStruct((N, 128), F32),
            jax.ShapeDtypeStruct((N, 1), F32),
            jax.ShapeDtypeStruct((N, 8), F32),
            jax.ShapeDtypeStruct((1, 1), F32),
        ],
        scratch_shapes=[pltpu.VMEM((1, 8), F32), pltpu.VMEM((1, 8), F32)],
    )(x, w_gate, degp)


def _fl_rest_body(sq_ref, ux_ref, dinv_ref, w0_ref, w1_ref, out_ref):
    dinv = dinv_ref[...]
    q = dinv * (sq_ref[0] + sq_ref[1] + ux_ref[...])
    h1 = jax.nn.relu(jnp.dot(q, w0_ref[0], preferred_element_type=F32))
    u = dinv * jnp.dot(h1, w1_ref[0], preferred_element_type=F32)
    out_ref[0, 0] = u[:, :128]
    out_ref[0, 1] = u[:, 128:]


def _fl_rest(sq, ux, dinv, w0s, w1s):
    return pl.pallas_call(
        _fl_rest_body,
        grid=(7, NBLK),
        in_specs=[
            pl.BlockSpec((2, BM, 128), lambda e, i: (0, i, 0)),
            pl.BlockSpec((BM, 128), lambda e, i: (i, 0)),
            pl.BlockSpec((BM, 1), lambda e, i: (i, 0)),
            pl.BlockSpec((1, 128, 256), lambda e, i: (e, 0, 0)),
            pl.BlockSpec((1, 256, 256), lambda e, i: (e, 0, 0)),
        ],
        out_specs=pl.BlockSpec((1, 2, BM, 128), lambda e, i: (e, 0, i, 0)),
        out_shape=jax.ShapeDtypeStruct((7, 2, N, 128), F32),
    )(sq, ux, dinv, w0s, w1s)


def _fl_e0_body(sq_ref, ux_ref, dinv_ref, w0_ref, w1_ref, out_ref):
    dinv = dinv_ref[...]
    q = dinv * (sq_ref[0] + sq_ref[1] + ux_ref[...])
    h1 = jax.nn.relu(jnp.dot(q, w0_ref[...], preferred_element_type=F32))
    out_ref[...] = dinv * jnp.dot(h1, w1_ref[...], preferred_element_type=F32)


def _fl_e0(sq, ux, dinv, w0, w1):
    return pl.pallas_call(
        _fl_e0_body,
        grid=(NBLK,),
        in_specs=[
            pl.BlockSpec((2, BM, 128), lambda i: (0, i, 0)),
            pl.BlockSpec((BM, 128), lambda i: (i, 0)),
            pl.BlockSpec((BM, 1), lambda i: (i, 0)),
            pl.BlockSpec((128, 256), lambda i: (0, 0)),
            pl.BlockSpec((256, 128), lambda i: (0, 0)),
        ],
        out_specs=pl.BlockSpec((BM, 128), lambda i: (i, 0)),
        out_shape=jax.ShapeDtypeStruct((N, 128), F32),
    )(sq, ux, dinv, w0, w1)


def _mid_body(s_ref, u_ref, dinv_ref, w_ref, out_ref):
    dinv = dinv_ref[...]
    t0 = jax.nn.relu(dinv * (s_ref[0] + u_ref[0]))
    t1 = jax.nn.relu(dinv * (s_ref[1] + u_ref[1]))
    w = w_ref[...]
    z = jnp.dot(t0, w[:128, :], preferred_element_type=F32)
    z = z + jnp.dot(t1, w[128:, :], preferred_element_type=F32)
    u = dinv * z
    out_ref[0] = u[:, :128]
    out_ref[1] = u[:, 128:]


def _mid(s, u, dinv, w):
    return pl.pallas_call(
        _mid_body,
        grid=(NBLK,),
        in_specs=[
            pl.BlockSpec((2, BM, 128), lambda i: (0, i, 0)),
            pl.BlockSpec((2, BM, 128), lambda i: (0, i, 0)),
            pl.BlockSpec((BM, 1), lambda i: (i, 0)),
            pl.BlockSpec((256, 256), lambda i: (0, 0)),
        ],
        out_specs=pl.BlockSpec((2, BM, 128), lambda i: (0, i, 0)),
        out_shape=jax.ShapeDtypeStruct((2, N, 128), F32),
    )(s, u, dinv, w)


def _last_body(s_ref, u_ref, dinv_ref, w_ref, out_ref):
    dinv = dinv_ref[...]
    t0 = jax.nn.relu(dinv * (s_ref[0] + u_ref[0]))
    t1 = jax.nn.relu(dinv * (s_ref[1] + u_ref[1]))
    w = w_ref[...]
    z = jnp.dot(t0, w[:128, :], preferred_element_type=F32)
    z = z + jnp.dot(t1, w[128:, :], preferred_element_type=F32)
    out_ref[...] = dinv * z


def _last(s, u, dinv, w):
    return pl.pallas_call(
        _last_body,
        grid=(NBLK,),
        in_specs=[
            pl.BlockSpec((2, BM, 128), lambda i: (0, i, 0)),
            pl.BlockSpec((2, BM, 128), lambda i: (0, i, 0)),
            pl.BlockSpec((BM, 1), lambda i: (i, 0)),
            pl.BlockSpec((256, 128), lambda i: (0, 0)),
        ],
        out_specs=pl.BlockSpec((BM, 128), lambda i: (i, 0)),
        out_shape=jax.ShapeDtypeStruct((N, 128), F32),
    )(s, u, dinv, w)


def _combine_body(gates_ref, dinv_ref, *refs):
    s_refs = refs[:8]
    u_refs = refs[8:16]
    y_ref = refs[16]
    dinv = dinv_ref[...]
    gates = gates_ref[...]
    y = jnp.zeros((BM, 128), F32)
    for e in range(8):
        out_e = dinv * (s_refs[e][0] + s_refs[e][1] + u_refs[e][...])
        y = y + gates[:, e:e + 1] * out_e
    y_ref[...] = y


def _combine(gates, dinv, s_list, u_list):
    return pl.pallas_call(
        _combine_body,
        grid=(NBLK,),
        in_specs=(
            [pl.BlockSpec((BM, 8), lambda i: (i, 0)),
             pl.BlockSpec((BM, 1), lambda i: (i, 0))]
            + [pl.BlockSpec((2, BM, 128), lambda i: (0, i, 0)) for _ in range(8)]
            + [pl.BlockSpec((BM, 128), lambda i: (i, 0)) for _ in range(8)]
        ),
        out_specs=pl.BlockSpec((BM, 128), lambda i: (i, 0)),
        out_shape=jax.ShapeDtypeStruct((N, 128), F32),
    )(gates, dinv, *s_list, *u_list)


def kernel(x, adj_t, w_gate, expert_weights):
    src = adj_t[0]
    dst = adj_t[1]
    pe = ED_GROUPS * NBUF * PCH - E // (NC * NS)
    pc = CH_GROUPS * NBUF * PCH - E // NS
    ps = (jnp.arange(pc, dtype=jnp.int32) * 131) % N
    pd = N + (jnp.arange(pc, dtype=jnp.int32) % 16)
    src_ed = jnp.concatenate(
        [src.reshape(NC * NS, -1), jnp.broadcast_to(ps[:pe], (NC * NS, pe))],
        axis=1).reshape(NC * NS, ED_GROUPS, NBUF, PCH)
    dst_ed = jnp.concatenate(
        [dst.reshape(NC * NS, -1), jnp.broadcast_to(pd[:pe], (NC * NS, pe))],
        axis=1).reshape(NC * NS, ED_GROUPS, NBUF, PCH)
    base_s = jnp.concatenate(
        [src.reshape(NS, -1), jnp.broadcast_to(ps, (NS, pc))], axis=1)
    src_ch = jnp.stack([base_s, base_s + N]).reshape(2, NS, CH_GROUPS, NBUF, PCH)
    dst_ch = jnp.concatenate(
        [dst.reshape(NS, -1), jnp.broadcast_to(pd, (NS, pc))], axis=1
    ).reshape(NS, CH_GROUPS, NBUF, PCH)

    degp = _deg_call(dst_ed)[:, :N].T
    ux, dinv, gates, loss11 = _prep(x, w_gate, degp)

    sq = _prop_ed(ux, src_ed, dst_ed)

    w0s = jnp.stack([expert_weights[e][0] for e in range(1, 8)])
    w1s = jnp.stack([expert_weights[e][1] for e in range(1, 8)])
    u1_rest = _fl_rest(sq, ux, dinv, w0s, w1s)
    u0 = _fl_e0(sq, ux, dinv, expert_weights[0][0], expert_weights[0][1])

    s_list = [None] * 8
    u_list = [None] * 8
    s_list[0] = _prop_ed(u0, src_ed, dst_ed)
    u_list[0] = u0
    for e in range(1, 8):
        L = 2 + e
        u = u1_rest[e - 1]
        ul = None
        for l in range(1, L - 1):
            sprop = _prop_ch(u.reshape(NC * N, 128), src_ch, dst_ch)
            w_next = expert_weights[e][l + 1]
            if l + 1 <= L - 2:
                u = _mid(sprop, u, dinv, w_next)
            else:
                ul = _last(sprop, u, dinv, w_next)
        s_list[e] = _prop_ed(ul, src_ed, dst_ed)
        u_list[e] = ul

    y = _combine(gates, dinv, s_list, u_list)
    return (y, loss11.reshape(()))
